# baseline (speedup 1.0000x reference)
import functools

import jax
import jax.numpy as jnp
from jax import lax
from jax.experimental import pallas as pl
from jax.experimental.pallas import tpu as pltpu
from jax.experimental.pallas import tpu_sc as plsc

L = 16
NP = 8732
NP_PAD = 8736
NCHUNK = NP_PAD // L
NOBJ = 20
NUM = 32
NEGPOS_RATIO = 3
THRESHOLD = 0.5

_LN2 = 0.6931471805599453
_SQRT2 = 1.4142135623730951


def _iota():
    return lax.iota(jnp.int32, L)


_LNC = (-0.10007079845408963, 0.1603383217490676, -0.17208502062121211,
        0.19920709285230567, -0.24974698884028282, 0.3333449009802525,
        -0.5000027504782988, 0.9999999700911615)


def _ln16(x):
    bits = lax.bitcast_convert_type(x, jnp.int32)
    e = (bits >> 23) - 127
    m = lax.bitcast_convert_type((bits & 0x007FFFFF) | 0x3F800000, jnp.float32)
    big = m > _SQRT2
    m = jnp.where(big, m * 0.5, m)
    e = jnp.where(big, e + 1, e)
    t = m - 1.0
    p = jnp.float32(_LNC[0])
    for c in _LNC[1:]:
        p = p * t + jnp.float32(c)
    return e.astype(jnp.float32) * jnp.float32(_LN2) + p * t


def _scalar(x):
    return jnp.max(x) if getattr(x, "ndim", 0) else x


def _sc_body(loc_hbm, conf_hbm, pri_hbm, tr_hbm, out_hbm,
             px1, py1, px2, py2,
             l0, l1, l2, l3, c0, c1,
             bi, bu, bti, keyr,
             tx1, ty1, tx2, ty2,
             mv, hist, outv, bp, ts, sem):
    wid = lax.axis_index("s") * 2 + lax.axis_index("c")
    iot = _iota()

    with jax.named_scope("sc_stage"):
        base4 = wid * 4
        base2 = wid * 2
        copies = (
            pltpu.make_async_copy(pri_hbm.at[0], px1, sem),
            pltpu.make_async_copy(pri_hbm.at[1], py1, sem),
            pltpu.make_async_copy(pri_hbm.at[2], px2, sem),
            pltpu.make_async_copy(pri_hbm.at[3], py2, sem),
            pltpu.make_async_copy(loc_hbm.at[base4 + 0], l0, sem),
            pltpu.make_async_copy(loc_hbm.at[base4 + 1], l1, sem),
            pltpu.make_async_copy(loc_hbm.at[base4 + 2], l2, sem),
            pltpu.make_async_copy(loc_hbm.at[base4 + 3], l3, sem),
            pltpu.make_async_copy(conf_hbm.at[base2 + 0], c0, sem),
            pltpu.make_async_copy(conf_hbm.at[base2 + 1], c1, sem),
            pltpu.make_async_copy(tr_hbm.at[base4 + 0], tx1, sem),
            pltpu.make_async_copy(tr_hbm.at[base4 + 1], ty1, sem),
            pltpu.make_async_copy(tr_hbm.at[base4 + 2], tx2, sem),
            pltpu.make_async_copy(tr_hbm.at[base4 + 3], ty2, sem),
        )
        for c in copies:
            c.start()
        for c in copies[:4]:
            c.wait()
        for c in copies[10:]:
            c.wait()

    with jax.named_scope("sc_p0"):
        def p0(ci, _):
            o = pl.multiple_of(ci * L, L)
            sl = pl.ds(o, L)
            cx, cy, w, h = px1[sl], py1[sl], px2[sl], py2[sl]
            px1[sl] = cx - 0.5 * w
            px2[sl] = cx + 0.5 * w
            py1[sl] = cy - 0.5 * h
            py2[sl] = cy + 0.5 * h
            return 0

        lax.fori_loop(0, NCHUNK, p0, 0)

        def tb_body(t, _):
            tvi = jnp.full((L,), t, jnp.int32)
            bx1 = plsc.load_gather(tx1, [tvi])
            by1 = plsc.load_gather(ty1, [tvi])
            bx2 = plsc.load_gather(tx2, [tvi])
            by2 = plsc.load_gather(ty2, [tvi])
            t6 = t * 6
            ts[t6 + 0] = jnp.max(bx1)
            ts[t6 + 1] = jnp.max(by1)
            ts[t6 + 2] = jnp.max(bx2)
            ts[t6 + 3] = jnp.max(by2)
            ts[t6 + 4] = jnp.max((bx2 - bx1) * (by2 - by1))
            o2 = pl.multiple_of(t * 3 * L, L)
            mv[pl.ds(o2, L)] = jnp.full((L,), -1.0, jnp.float32)
            mv[pl.ds(o2 + L, L)] = jnp.full((L,), 1.0, jnp.float32)
            mv[pl.ds(o2 + 2 * L, L)] = lax.bitcast_convert_type(
                jnp.zeros((L,), jnp.int32), jnp.float32)
            return 0

        lax.fori_loop(0, NOBJ, tb_body, 0)

    with jax.named_scope("sc_passA"):
        def chunk_body(ci, _):
            o = pl.multiple_of(ci * L, L)
            sl = pl.ds(o, L)
            x1, y1, x2, y2 = px1[sl], py1[sl], px2[sl], py2[sl]
            ap = (x2 - x1) * (y2 - y1)
            gidx = o + iot
            invalid = gidx >= NP
            b_i = jnp.zeros((L,), jnp.float32)
            b_u = jnp.full((L,), 1.0, jnp.float32)
            b_t = jnp.zeros((L,), jnp.int32)
            for t in range(NOBJ):
                t6 = t * 6
                bx1 = ts[t6 + 0]
                by1 = ts[t6 + 1]
                bx2 = ts[t6 + 2]
                by2 = ts[t6 + 3]
                at = ts[t6 + 4]
                iw = jnp.maximum(
                    jnp.minimum(x2, bx2) - jnp.maximum(x1, bx1), 0.0)
                ih = jnp.maximum(
                    jnp.minimum(y2, by2) - jnp.maximum(y1, by1), 0.0)
                inter = jnp.where(invalid, -1.0, iw * ih)
                union = at + ap - inter
                upd = inter * b_u > b_i * union
                b_i = jnp.where(upd, inter, b_i)
                b_u = jnp.where(upd, union, b_u)
                b_t = jnp.where(upd, t, b_t)
                o2 = t * 3 * L
                li = mv[pl.ds(o2, L)]
                lu = mv[pl.ds(o2 + L, L)]
                lidx = lax.bitcast_convert_type(mv[pl.ds(o2 + 2 * L, L)],
                                                jnp.int32)
                upd2 = inter * lu > li * union
                mv[pl.ds(o2, L)] = jnp.where(upd2, inter, li)
                mv[pl.ds(o2 + L, L)] = jnp.where(upd2, union, lu)
                mv[pl.ds(o2 + 2 * L, L)] = lax.bitcast_convert_type(
                    jnp.where(upd2, gidx, lidx), jnp.float32)
            bi[sl] = b_i
            bu[sl] = b_u
            bti[sl] = b_t
            return 0

        lax.fori_loop(0, NCHUNK, chunk_body, 0)

    with jax.named_scope("sc_bp_force"):
        def bp_body(t, _):
            o2 = pl.multiple_of(t * 3 * L, L)
            li = mv[pl.ds(o2, L)]
            lu = mv[pl.ds(o2 + L, L)]
            lidx = lax.bitcast_convert_type(mv[pl.ds(o2 + 2 * L, L)],
                                            jnp.int32)
            q = li / lu
            m = jnp.max(q)
            cand = jnp.where(q == m, lidx, NP_PAD)
            bp[t] = jnp.min(cand)
            return 0

        lax.fori_loop(0, NOBJ, bp_body, 0)

        def force_body(t, _):
            j = bp[t]
            o = pl.multiple_of((j >> 4) << 4, L)
            lane = j & 15
            sl = pl.ds(o, L)
            hit = iot == lane
            bi[sl] = jnp.where(hit, 2.0, bi[sl])
            bu[sl] = jnp.where(hit, 1.0, bu[sl])
            bti[sl] = jnp.where(hit, t, bti[sl])
            return 0

        lax.fori_loop(0, NOBJ, force_body, 0)

        for j in range(L):
            hist[pl.ds(j * L, L)] = jnp.zeros((L,), jnp.int32)

    ones = jnp.full((L,), 1, jnp.int32)

    with jax.named_scope("sc_wait_lc"):
        for c in copies[4:10]:
            c.wait()

    with jax.named_scope("sc_passC"):
        def main_body(ci, carry):
            acc_l, acc_c, npos = carry
            o = pl.multiple_of(ci * L, L)
            sl = pl.ds(o, L)
            pos = 2.0 * bi[sl] >= bu[sl]
            bt = bti[sl]
            mx1 = plsc.load_gather(tx1, [bt])
            my1 = plsc.load_gather(ty1, [bt])
            mx2 = plsc.load_gather(tx2, [bt])
            my2 = plsc.load_gather(ty2, [bt])
            x1, y1, x2, y2 = px1[sl], py1[sl], px2[sl], py2[sl]
            cx = (x1 + x2) * 0.5
            cy = (y1 + y2) * 0.5
            w = x2 - x1
            h = y2 - y1
            gcx = ((mx1 + mx2) * 0.5 - cx) / (0.1 * w)
            gcy = ((my1 + my2) * 0.5 - cy) / (0.1 * h)
            gw = _ln16((mx2 - mx1) / w) * 5.0
            gh = _ln16((my2 - my1) / h) * 5.0
            sm = jnp.zeros((L,), jnp.float32)
            for lref, g in ((l0, gcx), (l1, gcy), (l2, gw), (l3, gh)):
                d = lref[sl] - g
                ad = jnp.abs(d)
                sm = sm + jnp.where(ad < 1.0, 0.5 * d * d, ad - 0.5)
            acc_l = acc_l + jnp.where(pos, sm, 0.0)
            npos = npos + jnp.where(pos, 1, 0)
            x0 = c0[sl]
            xo = c1[sl]
            mxx = jnp.maximum(x0, xo)
            mnn = jnp.minimum(x0, xo)
            lse = _ln16(1.0 + jnp.exp(mnn - mxx)) + mxx
            xt = jnp.where(pos, xo, x0)
            ce = lse - xt
            acc_c = acc_c + jnp.where(pos, ce, 0.0)
            gidx = o + iot
            keyf = jnp.where(jnp.logical_and(gidx < NP, jnp.logical_not(pos)),
                             ce, 0.0)
            kb = lax.bitcast_convert_type(keyf, jnp.int32)
            keyr[sl] = kb
            plsc.addupdate_scatter(hist, [(kb >> 24) & 255], ones)
            return acc_l, acc_c, npos

        acc_l, acc_c, npos = lax.fori_loop(
            0, NCHUNK, main_body,
            (jnp.zeros((L,), jnp.float32), jnp.zeros((L,), jnp.float32),
             jnp.zeros((L,), jnp.int32)))
        num_pos = jnp.sum(npos)
        loss_l_s = jnp.sum(acc_l)
        ce_pos_s = jnp.sum(acc_c)
        k = jnp.minimum(NEGPOS_RATIO * num_pos, NP - 1)

    with jax.named_scope("sc_radix"):
        UNROLL = 6
        prefix = jnp.int32(0)
        c_before = jnp.int32(0)
        for rnd, d in enumerate((24, 16, 8, 0)):
            if rnd > 0:
                for j in range(L):
                    hist[pl.ds(j * L, L)] = jnp.zeros((L,), jnp.int32)

                def hsweep(cg, _, d=d, prefix=prefix):
                    o0 = pl.multiple_of(cg * (L * UNROLL), L)
                    for u in range(UNROLL):
                        kb = keyr[pl.ds(o0 + u * L, L)]
                        digit = (kb >> d) & 255
                        region = (kb >> (d + 8)) == prefix
                        plsc.addupdate_scatter(hist, [digit], ones,
                                               mask=region)
                    return 0

                lax.fori_loop(0, NCHUNK // UNROLL, hsweep, 0)

            csum = jnp.zeros((L,), jnp.int32)
            for c in range(L):
                s = jnp.sum(hist[pl.ds(c * L, L)])
                csum = jnp.where(iot == c, s, csum)
            rs = jnp.flip(csum, axis=0)
            cs = plsc.cumsum(rs)
            k_rel = k - c_before
            jj = _scalar(plsc.all_reduce_ffs(cs >= k_rel))
            gt_chunks = jnp.sum(jnp.where(iot < jj, rs, 0))
            cstar = 15 - jj
            hc = hist[pl.ds(pl.multiple_of(cstar * L, L), L)]
            rhc = jnp.flip(hc, axis=0)
            cs2 = plsc.cumsum(rhc)
            jj2 = _scalar(plsc.all_reduce_ffs(cs2 >= (k_rel - gt_chunks)))
            gt_in = jnp.sum(jnp.where(iot < jj2, rhc, 0))
            digit = cstar * L + (15 - jj2)
            c_before = c_before + gt_chunks + gt_in
            prefix = digit if rnd == 0 else (prefix << 8) | digit

        v = prefix
        c_gt = c_before

    with jax.named_scope("sc_passE"):
        def esweep(cg, acc):
            o0 = pl.multiple_of(cg * (L * UNROLL), L)
            for u in range(UNROLL):
                kb = keyr[pl.ds(o0 + u * L, L)]
                keyf = lax.bitcast_convert_type(kb, jnp.float32)
                acc = acc + jnp.where(kb > v, keyf, 0.0)
            return acc

        acc_n = lax.fori_loop(0, NCHUNK // UNROLL, esweep,
                              jnp.zeros((L,), jnp.float32))
        vf = lax.bitcast_convert_type(v, jnp.float32)
        loss_c_s = (ce_pos_s + jnp.sum(acc_n)
                    + (k - c_gt).astype(jnp.float32) * vf)

        outv[...] = jnp.where(
            iot == 0, loss_l_s,
            jnp.where(iot == 1, loss_c_s,
                      jnp.where(iot == 2, num_pos.astype(jnp.float32), 0.0)))
        pltpu.sync_copy(outv, out_hbm.at[wid])


@functools.partial(
    pl.kernel,
    mesh=plsc.VectorSubcoreMesh(core_axis_name="c", subcore_axis_name="s"),
    out_type=jax.ShapeDtypeStruct((NUM, L), jnp.float32),
    compiler_params=pltpu.CompilerParams(needs_layout_passes=False),
    scratch_types=[
        pltpu.VMEM((NP_PAD,), jnp.float32),
        pltpu.VMEM((NP_PAD,), jnp.float32),
        pltpu.VMEM((NP_PAD,), jnp.float32),
        pltpu.VMEM((NP_PAD,), jnp.float32),
        pltpu.VMEM((NP_PAD,), jnp.float32),
        pltpu.VMEM((NP_PAD,), jnp.float32),
        pltpu.VMEM((NP_PAD,), jnp.float32),
        pltpu.VMEM((NP_PAD,), jnp.float32),
        pltpu.VMEM((NP_PAD,), jnp.float32),
        pltpu.VMEM((NP_PAD,), jnp.float32),
        pltpu.VMEM((NP_PAD,), jnp.float32),
        pltpu.VMEM((NP_PAD,), jnp.float32),
        pltpu.VMEM((NP_PAD,), jnp.int32),
        pltpu.VMEM((NP_PAD,), jnp.int32),
        pltpu.VMEM((32,), jnp.float32),
        pltpu.VMEM((32,), jnp.float32),
        pltpu.VMEM((32,), jnp.float32),
        pltpu.VMEM((32,), jnp.float32),
        pltpu.VMEM((NOBJ * 3 * L,), jnp.float32),
        pltpu.VMEM((256,), jnp.int32),
        pltpu.VMEM((L,), jnp.float32),
        pltpu.SMEM((32,), jnp.int32),
        pltpu.SMEM((NOBJ * 6,), jnp.float32),
        pltpu.SemaphoreType.DMA,
    ],
)
def _sc_loss(loc_hbm, conf_hbm, pri_hbm, tr_hbm, out_hbm, *scratch):
    _sc_body(loc_hbm, conf_hbm, pri_hbm, tr_hbm, out_hbm, *scratch)


def kernel(loc_data, conf_data, priors, targets):
    num, np_, _ = loc_data.shape
    pad = NP_PAD - np_
    loc_r = jnp.transpose(loc_data, (0, 2, 1))
    loc_r = jnp.pad(loc_r, ((0, 0), (0, 0), (0, pad))).reshape(num * 4, NP_PAD)
    conf_r = jnp.transpose(conf_data, (0, 2, 1))
    conf_r = jnp.pad(conf_r, ((0, 0), (0, 0), (0, pad))).reshape(num * 2, NP_PAD)
    pri_r = jnp.transpose(priors[:np_], (1, 0))
    pri_r = jnp.pad(pri_r, ((0, 0), (0, pad)), constant_values=1.0)
    tr_r = jnp.transpose(targets[:, :, :4], (0, 2, 1))
    tr_r = jnp.pad(tr_r, ((0, 0), (0, 0), (0, 32 - tr_r.shape[2])))
    tr_r = tr_r.reshape(num * 4, 32)
    part = _sc_loss(loc_r, conf_r, pri_r, tr_r)
    n = jnp.sum(part[:, 2])
    return jnp.sum(part[:, 0]) / n, jnp.sum(part[:, 1]) / n


# speedup vs baseline: 1.4201x; 1.4201x over previous
import functools

import jax
import jax.numpy as jnp
from jax import lax
from jax.experimental import pallas as pl
from jax.experimental.pallas import tpu as pltpu
from jax.experimental.pallas import tpu_sc as plsc

L = 16
NP = 8732
NP_PAD = 8736
NCHUNK = NP_PAD // L
NOBJ = 20
NUM = 32
NEGPOS_RATIO = 3
THRESHOLD = 0.5

_LN2 = 0.6931471805599453
_SQRT2 = 1.4142135623730951


def _iota():
    return lax.iota(jnp.int32, L)


_LNC = (-0.10007079845408963, 0.1603383217490676, -0.17208502062121211,
        0.19920709285230567, -0.24974698884028282, 0.3333449009802525,
        -0.5000027504782988, 0.9999999700911615)


def _ln16(x):
    bits = lax.bitcast_convert_type(x, jnp.int32)
    e = (bits >> 23) - 127
    m = lax.bitcast_convert_type((bits & 0x007FFFFF) | 0x3F800000, jnp.float32)
    big = m > _SQRT2
    m = jnp.where(big, m * 0.5, m)
    e = jnp.where(big, e + 1, e)
    t = m - 1.0
    p = jnp.float32(_LNC[0])
    for c in _LNC[1:]:
        p = p * t + jnp.float32(c)
    return e.astype(jnp.float32) * jnp.float32(_LN2) + p * t


def _scalar(x):
    return jnp.max(x) if getattr(x, "ndim", 0) else x


def _sc_body(loc_hbm, conf_hbm, pri_hbm, tr_hbm, out_hbm,
             px1, py1, px2, py2,
             l0, l1, l2, l3, c0, c1,
             bi, bu, bti, keyr,
             tx1, ty1, tx2, ty2,
             tb, mv, hist, outv, bp, sem):
    wid = lax.axis_index("s") * 2 + lax.axis_index("c")
    iot = _iota()

    with jax.named_scope("sc_stage"):
        base4 = wid * 4
        base2 = wid * 2
        copies = (
            pltpu.make_async_copy(pri_hbm.at[0], px1, sem),
            pltpu.make_async_copy(pri_hbm.at[1], py1, sem),
            pltpu.make_async_copy(pri_hbm.at[2], px2, sem),
            pltpu.make_async_copy(pri_hbm.at[3], py2, sem),
            pltpu.make_async_copy(loc_hbm.at[base4 + 0], l0, sem),
            pltpu.make_async_copy(loc_hbm.at[base4 + 1], l1, sem),
            pltpu.make_async_copy(loc_hbm.at[base4 + 2], l2, sem),
            pltpu.make_async_copy(loc_hbm.at[base4 + 3], l3, sem),
            pltpu.make_async_copy(conf_hbm.at[base2 + 0], c0, sem),
            pltpu.make_async_copy(conf_hbm.at[base2 + 1], c1, sem),
            pltpu.make_async_copy(tr_hbm.at[base4 + 0], tx1, sem),
            pltpu.make_async_copy(tr_hbm.at[base4 + 1], ty1, sem),
            pltpu.make_async_copy(tr_hbm.at[base4 + 2], tx2, sem),
            pltpu.make_async_copy(tr_hbm.at[base4 + 3], ty2, sem),
        )
        for c in copies:
            c.start()
        for c in copies[:4]:
            c.wait()
        for c in copies[10:]:
            c.wait()

    with jax.named_scope("sc_p0"):
        def p0(ci, _):
            o = pl.multiple_of(ci * L, L)
            sl = pl.ds(o, L)
            cx, cy, w, h = px1[sl], py1[sl], px2[sl], py2[sl]
            px1[sl] = cx - 0.5 * w
            px2[sl] = cx + 0.5 * w
            py1[sl] = cy - 0.5 * h
            py2[sl] = cy + 0.5 * h
            return 0

        lax.fori_loop(0, NCHUNK, p0, 0)

        def tb_body(t, _):
            tvi = jnp.full((L,), t, jnp.int32)
            bx1 = plsc.load_gather(tx1, [tvi])
            by1 = plsc.load_gather(ty1, [tvi])
            bx2 = plsc.load_gather(tx2, [tvi])
            by2 = plsc.load_gather(ty2, [tvi])
            o = pl.multiple_of(t * 5 * L, L)
            tb[pl.ds(o, L)] = bx1
            tb[pl.ds(o + L, L)] = by1
            tb[pl.ds(o + 2 * L, L)] = bx2
            tb[pl.ds(o + 3 * L, L)] = by2
            tb[pl.ds(o + 4 * L, L)] = (bx2 - bx1) * (by2 - by1)
            o2 = pl.multiple_of(t * 3 * L, L)
            mv[pl.ds(o2, L)] = jnp.full((L,), -1.0, jnp.float32)
            mv[pl.ds(o2 + L, L)] = jnp.full((L,), 1.0, jnp.float32)
            mv[pl.ds(o2 + 2 * L, L)] = lax.bitcast_convert_type(
                jnp.zeros((L,), jnp.int32), jnp.float32)
            return 0

        lax.fori_loop(0, NOBJ, tb_body, 0)

    with jax.named_scope("sc_passA"):
        def chunk_body(ci, _):
            o = pl.multiple_of(ci * L, L)
            sl = pl.ds(o, L)
            x1, y1, x2, y2 = px1[sl], py1[sl], px2[sl], py2[sl]
            ap = (x2 - x1) * (y2 - y1)
            gidx = o + iot
            invalid = gidx >= NP
            b_i = jnp.zeros((L,), jnp.float32)
            b_u = jnp.full((L,), 1.0, jnp.float32)
            b_t = jnp.zeros((L,), jnp.int32)
            for t in range(NOBJ):
                ob = t * 5 * L
                bx1 = tb[pl.ds(ob, L)]
                by1 = tb[pl.ds(ob + L, L)]
                bx2 = tb[pl.ds(ob + 2 * L, L)]
                by2 = tb[pl.ds(ob + 3 * L, L)]
                at = tb[pl.ds(ob + 4 * L, L)]
                iw = jnp.maximum(
                    jnp.minimum(x2, bx2) - jnp.maximum(x1, bx1), 0.0)
                ih = jnp.maximum(
                    jnp.minimum(y2, by2) - jnp.maximum(y1, by1), 0.0)
                inter = jnp.where(invalid, -1.0, iw * ih)
                union = at + ap - inter
                upd = inter * b_u > b_i * union
                b_i = jnp.where(upd, inter, b_i)
                b_u = jnp.where(upd, union, b_u)
                b_t = jnp.where(upd, t, b_t)
                o2 = t * 3 * L
                li = mv[pl.ds(o2, L)]
                lu = mv[pl.ds(o2 + L, L)]
                lidx = lax.bitcast_convert_type(mv[pl.ds(o2 + 2 * L, L)],
                                                jnp.int32)
                upd2 = inter * lu > li * union
                mv[pl.ds(o2, L)] = jnp.where(upd2, inter, li)
                mv[pl.ds(o2 + L, L)] = jnp.where(upd2, union, lu)
                mv[pl.ds(o2 + 2 * L, L)] = lax.bitcast_convert_type(
                    jnp.where(upd2, gidx, lidx), jnp.float32)
            bi[sl] = b_i
            bu[sl] = b_u
            bti[sl] = b_t
            return 0

        lax.fori_loop(0, NCHUNK, chunk_body, 0)

    with jax.named_scope("sc_bp_force"):
        def bp_body(t, _):
            o2 = pl.multiple_of(t * 3 * L, L)
            li = mv[pl.ds(o2, L)]
            lu = mv[pl.ds(o2 + L, L)]
            lidx = lax.bitcast_convert_type(mv[pl.ds(o2 + 2 * L, L)],
                                            jnp.int32)
            q = li / lu
            m = jnp.max(q)
            cand = jnp.where(q == m, lidx, NP_PAD)
            bp[t] = jnp.min(cand)
            return 0

        lax.fori_loop(0, NOBJ, bp_body, 0)

        def force_body(t, _):
            j = bp[t]
            o = pl.multiple_of((j >> 4) << 4, L)
            lane = j & 15
            sl = pl.ds(o, L)
            hit = iot == lane
            bi[sl] = jnp.where(hit, 2.0, bi[sl])
            bu[sl] = jnp.where(hit, 1.0, bu[sl])
            bti[sl] = jnp.where(hit, t, bti[sl])
            return 0

        lax.fori_loop(0, NOBJ, force_body, 0)

        for j in range(2 * L):
            hist[pl.ds(j * L, L)] = jnp.zeros((L,), jnp.int32)

    ones = jnp.full((L,), 1, jnp.int32)

    with jax.named_scope("sc_wait_lc"):
        for c in copies[4:10]:
            c.wait()

    with jax.named_scope("sc_passC"):
        def main_body(ci, carry):
            acc_l, acc_c, npos = carry
            o = pl.multiple_of(ci * L, L)
            sl = pl.ds(o, L)
            pos = 2.0 * bi[sl] >= bu[sl]
            bt = bti[sl]
            mx1 = plsc.load_gather(tx1, [bt])
            my1 = plsc.load_gather(ty1, [bt])
            mx2 = plsc.load_gather(tx2, [bt])
            my2 = plsc.load_gather(ty2, [bt])
            x1, y1, x2, y2 = px1[sl], py1[sl], px2[sl], py2[sl]
            cx = (x1 + x2) * 0.5
            cy = (y1 + y2) * 0.5
            w = x2 - x1
            h = y2 - y1
            gcx = ((mx1 + mx2) * 0.5 - cx) / (0.1 * w)
            gcy = ((my1 + my2) * 0.5 - cy) / (0.1 * h)
            gw = _ln16((mx2 - mx1) / w) * 5.0
            gh = _ln16((my2 - my1) / h) * 5.0
            sm = jnp.zeros((L,), jnp.float32)
            for lref, g in ((l0, gcx), (l1, gcy), (l2, gw), (l3, gh)):
                d = lref[sl] - g
                ad = jnp.abs(d)
                sm = sm + jnp.where(ad < 1.0, 0.5 * d * d, ad - 0.5)
            acc_l = acc_l + jnp.where(pos, sm, 0.0)
            npos = npos + jnp.where(pos, 1, 0)
            x0 = c0[sl]
            xo = c1[sl]
            mxx = jnp.maximum(x0, xo)
            mnn = jnp.minimum(x0, xo)
            lse = _ln16(1.0 + jnp.exp(mnn - mxx)) + mxx
            xt = jnp.where(pos, xo, x0)
            ce = lse - xt
            acc_c = acc_c + jnp.where(pos, ce, 0.0)
            gidx = o + iot
            keyf = jnp.where(jnp.logical_and(gidx < NP, jnp.logical_not(pos)),
                             ce, 0.0)
            kb = lax.bitcast_convert_type(keyf, jnp.int32)
            keyr[sl] = kb
            off = (ci & 1) << 8
            plsc.addupdate_scatter(hist, [((kb >> 24) & 255) + off], ones)
            return acc_l, acc_c, npos

        acc_l, acc_c, npos = lax.fori_loop(
            0, NCHUNK, main_body,
            (jnp.zeros((L,), jnp.float32), jnp.zeros((L,), jnp.float32),
             jnp.zeros((L,), jnp.int32)))
        num_pos = jnp.sum(npos)
        loss_l_s = jnp.sum(acc_l)
        ce_pos_s = jnp.sum(acc_c)
        k = jnp.minimum(NEGPOS_RATIO * num_pos, NP - 1)

    with jax.named_scope("sc_radix"):
        UNROLL = 6
        prefix = jnp.int32(0)
        c_before = jnp.int32(0)
        for rnd, d in enumerate((24, 16, 8, 0)):
            if rnd > 0:
                for j in range(2 * L):
                    hist[pl.ds(j * L, L)] = jnp.zeros((L,), jnp.int32)

                def hsweep(cg, _, d=d, prefix=prefix):
                    o0 = pl.multiple_of(cg * (L * UNROLL), L)
                    for u in range(UNROLL):
                        kb = keyr[pl.ds(o0 + u * L, L)]
                        digit = ((kb >> d) & 255) + ((u & 1) << 8)
                        region = (kb >> (d + 8)) == prefix
                        plsc.addupdate_scatter(hist, [digit], ones,
                                               mask=region)
                    return 0

                lax.fori_loop(0, NCHUNK // UNROLL, hsweep, 0)

            csum = jnp.zeros((L,), jnp.int32)
            for c in range(L):
                s = jnp.sum(hist[pl.ds(c * L, L)]
                            + hist[pl.ds(256 + c * L, L)])
                csum = jnp.where(iot == c, s, csum)
            rs = jnp.flip(csum, axis=0)
            cs = plsc.cumsum(rs)
            k_rel = k - c_before
            jj = _scalar(plsc.all_reduce_ffs(cs >= k_rel))
            gt_chunks = jnp.sum(jnp.where(iot < jj, rs, 0))
            cstar = 15 - jj
            hc = (hist[pl.ds(pl.multiple_of(cstar * L, L), L)]
                  + hist[pl.ds(pl.multiple_of(256 + cstar * L, L), L)])
            rhc = jnp.flip(hc, axis=0)
            cs2 = plsc.cumsum(rhc)
            jj2 = _scalar(plsc.all_reduce_ffs(cs2 >= (k_rel - gt_chunks)))
            gt_in = jnp.sum(jnp.where(iot < jj2, rhc, 0))
            digit = cstar * L + (15 - jj2)
            c_before = c_before + gt_chunks + gt_in
            prefix = digit if rnd == 0 else (prefix << 8) | digit

        v = prefix
        c_gt = c_before

    with jax.named_scope("sc_passE"):
        def esweep(cg, acc):
            o0 = pl.multiple_of(cg * (L * UNROLL), L)
            for u in range(UNROLL):
                kb = keyr[pl.ds(o0 + u * L, L)]
                keyf = lax.bitcast_convert_type(kb, jnp.float32)
                acc = acc + jnp.where(kb > v, keyf, 0.0)
            return acc

        acc_n = lax.fori_loop(0, NCHUNK // UNROLL, esweep,
                              jnp.zeros((L,), jnp.float32))
        vf = lax.bitcast_convert_type(v, jnp.float32)
        loss_c_s = (ce_pos_s + jnp.sum(acc_n)
                    + (k - c_gt).astype(jnp.float32) * vf)

        outv[...] = jnp.where(
            iot == 0, loss_l_s,
            jnp.where(iot == 1, loss_c_s,
                      jnp.where(iot == 2, num_pos.astype(jnp.float32), 0.0)))
        pltpu.sync_copy(outv, out_hbm.at[wid])


@functools.partial(
    pl.kernel,
    mesh=plsc.VectorSubcoreMesh(core_axis_name="c", subcore_axis_name="s"),
    out_type=jax.ShapeDtypeStruct((NUM, L), jnp.float32),
    compiler_params=pltpu.CompilerParams(needs_layout_passes=False),
    scratch_types=[
        pltpu.VMEM((NP_PAD,), jnp.float32),
        pltpu.VMEM((NP_PAD,), jnp.float32),
        pltpu.VMEM((NP_PAD,), jnp.float32),
        pltpu.VMEM((NP_PAD,), jnp.float32),
        pltpu.VMEM((NP_PAD,), jnp.float32),
        pltpu.VMEM((NP_PAD,), jnp.float32),
        pltpu.VMEM((NP_PAD,), jnp.float32),
        pltpu.VMEM((NP_PAD,), jnp.float32),
        pltpu.VMEM((NP_PAD,), jnp.float32),
        pltpu.VMEM((NP_PAD,), jnp.float32),
        pltpu.VMEM((NP_PAD,), jnp.float32),
        pltpu.VMEM((NP_PAD,), jnp.float32),
        pltpu.VMEM((NP_PAD,), jnp.int32),
        pltpu.VMEM((NP_PAD,), jnp.int32),
        pltpu.VMEM((32,), jnp.float32),
        pltpu.VMEM((32,), jnp.float32),
        pltpu.VMEM((32,), jnp.float32),
        pltpu.VMEM((32,), jnp.float32),
        pltpu.VMEM((NOBJ * 5 * L,), jnp.float32),
        pltpu.VMEM((NOBJ * 3 * L,), jnp.float32),
        pltpu.VMEM((512,), jnp.int32),
        pltpu.VMEM((L,), jnp.float32),
        pltpu.SMEM((32,), jnp.int32),
        pltpu.SemaphoreType.DMA,
    ],
)
def _sc_loss(loc_hbm, conf_hbm, pri_hbm, tr_hbm, out_hbm, *scratch):
    _sc_body(loc_hbm, conf_hbm, pri_hbm, tr_hbm, out_hbm, *scratch)


def kernel(loc_data, conf_data, priors, targets):
    num, np_, _ = loc_data.shape
    pad = NP_PAD - np_
    loc_r = jnp.transpose(loc_data, (0, 2, 1))
    loc_r = jnp.pad(loc_r, ((0, 0), (0, 0), (0, pad))).reshape(num * 4, NP_PAD)
    conf_r = jnp.transpose(conf_data, (0, 2, 1))
    conf_r = jnp.pad(conf_r, ((0, 0), (0, 0), (0, pad))).reshape(num * 2, NP_PAD)
    pri_r = jnp.transpose(priors[:np_], (1, 0))
    pri_r = jnp.pad(pri_r, ((0, 0), (0, pad)), constant_values=1.0)
    tr_r = jnp.transpose(targets[:, :, :4], (0, 2, 1))
    tr_r = jnp.pad(tr_r, ((0, 0), (0, 0), (0, 32 - tr_r.shape[2])))
    tr_r = tr_r.reshape(num * 4, 32)
    part = _sc_loss(loc_r, conf_r, pri_r, tr_r)
    n = jnp.sum(part[:, 2])
    return jnp.sum(part[:, 0]) / n, jnp.sum(part[:, 1]) / n


# speedup vs baseline: 1.4231x; 1.0021x over previous
import functools

import jax
import jax.numpy as jnp
from jax import lax
from jax.experimental import pallas as pl
from jax.experimental.pallas import tpu as pltpu
from jax.experimental.pallas import tpu_sc as plsc

L = 16
NP = 8732
NP_PAD = 8736
NCHUNK = NP_PAD // L
NOBJ = 20
NUM = 32
NEGPOS_RATIO = 3
THRESHOLD = 0.5

_LN2 = 0.6931471805599453
_SQRT2 = 1.4142135623730951


def _iota():
    return lax.iota(jnp.int32, L)


_LNC = (-0.10007079845408963, 0.1603383217490676, -0.17208502062121211,
        0.19920709285230567, -0.24974698884028282, 0.3333449009802525,
        -0.5000027504782988, 0.9999999700911615)


def _ln16(x):
    bits = lax.bitcast_convert_type(x, jnp.int32)
    e = (bits >> 23) - 127
    m = lax.bitcast_convert_type((bits & 0x007FFFFF) | 0x3F800000, jnp.float32)
    big = m > _SQRT2
    m = jnp.where(big, m * 0.5, m)
    e = jnp.where(big, e + 1, e)
    t = m - 1.0
    p = jnp.float32(_LNC[0])
    for c in _LNC[1:]:
        p = p * t + jnp.float32(c)
    return e.astype(jnp.float32) * jnp.float32(_LN2) + p * t


def _scalar(x):
    return jnp.max(x) if getattr(x, "ndim", 0) else x


def _sc_body(loc_hbm, conf_hbm, pri_hbm, tr_hbm, out_hbm,
             px1, py1, px2, py2,
             l0, l1, l2, l3, c0, c1,
             bi, bu, bti, keyr,
             tx1, ty1, tx2, ty2,
             tb, mv, hist, outv, bp, sem):
    wid = lax.axis_index("s") * 2 + lax.axis_index("c")
    iot = _iota()

    with jax.named_scope("sc_stage"):
        base4 = wid * 4
        base2 = wid * 2
        copies = (
            pltpu.make_async_copy(pri_hbm.at[0], px1, sem),
            pltpu.make_async_copy(pri_hbm.at[1], py1, sem),
            pltpu.make_async_copy(pri_hbm.at[2], px2, sem),
            pltpu.make_async_copy(pri_hbm.at[3], py2, sem),
            pltpu.make_async_copy(loc_hbm.at[base4 + 0], l0, sem),
            pltpu.make_async_copy(loc_hbm.at[base4 + 1], l1, sem),
            pltpu.make_async_copy(loc_hbm.at[base4 + 2], l2, sem),
            pltpu.make_async_copy(loc_hbm.at[base4 + 3], l3, sem),
            pltpu.make_async_copy(conf_hbm.at[base2 + 0], c0, sem),
            pltpu.make_async_copy(conf_hbm.at[base2 + 1], c1, sem),
            pltpu.make_async_copy(tr_hbm.at[base4 + 0], tx1, sem),
            pltpu.make_async_copy(tr_hbm.at[base4 + 1], ty1, sem),
            pltpu.make_async_copy(tr_hbm.at[base4 + 2], tx2, sem),
            pltpu.make_async_copy(tr_hbm.at[base4 + 3], ty2, sem),
        )
        for c in copies:
            c.start()
        for c in copies[:4]:
            c.wait()
        for c in copies[10:]:
            c.wait()

    with jax.named_scope("sc_p0"):
        def p0(ci, _):
            o = pl.multiple_of(ci * L, L)
            sl = pl.ds(o, L)
            cx, cy, w, h = px1[sl], py1[sl], px2[sl], py2[sl]
            px1[sl] = cx - 0.5 * w
            px2[sl] = cx + 0.5 * w
            py1[sl] = cy - 0.5 * h
            py2[sl] = cy + 0.5 * h
            return 0

        lax.fori_loop(0, NCHUNK, p0, 0)

        def tb_body(t, _):
            tvi = jnp.full((L,), t, jnp.int32)
            bx1 = plsc.load_gather(tx1, [tvi])
            by1 = plsc.load_gather(ty1, [tvi])
            bx2 = plsc.load_gather(tx2, [tvi])
            by2 = plsc.load_gather(ty2, [tvi])
            o = pl.multiple_of(t * 5 * L, L)
            tb[pl.ds(o, L)] = bx1
            tb[pl.ds(o + L, L)] = by1
            tb[pl.ds(o + 2 * L, L)] = bx2
            tb[pl.ds(o + 3 * L, L)] = by2
            tb[pl.ds(o + 4 * L, L)] = (bx2 - bx1) * (by2 - by1)
            o2 = pl.multiple_of(t * 3 * L, L)
            mv[pl.ds(o2, L)] = jnp.full((L,), -1.0, jnp.float32)
            mv[pl.ds(o2 + L, L)] = jnp.full((L,), 1.0, jnp.float32)
            mv[pl.ds(o2 + 2 * L, L)] = lax.bitcast_convert_type(
                jnp.zeros((L,), jnp.int32), jnp.float32)
            return 0

        lax.fori_loop(0, NOBJ, tb_body, 0)

    with jax.named_scope("sc_passA"):
        def chunk_body(ci, _):
            o = pl.multiple_of(ci * L, L)
            sl = pl.ds(o, L)
            x1, y1, x2, y2 = px1[sl], py1[sl], px2[sl], py2[sl]
            ap = (x2 - x1) * (y2 - y1)
            gidx = o + iot
            invalid = gidx >= NP
            b_i = jnp.zeros((L,), jnp.float32)
            b_u = jnp.full((L,), 1.0, jnp.float32)
            b_t = jnp.zeros((L,), jnp.int32)
            for t in range(NOBJ):
                ob = t * 5 * L
                bx1 = tb[pl.ds(ob, L)]
                by1 = tb[pl.ds(ob + L, L)]
                bx2 = tb[pl.ds(ob + 2 * L, L)]
                by2 = tb[pl.ds(ob + 3 * L, L)]
                at = tb[pl.ds(ob + 4 * L, L)]
                iw = jnp.maximum(
                    jnp.minimum(x2, bx2) - jnp.maximum(x1, bx1), 0.0)
                ih = jnp.maximum(
                    jnp.minimum(y2, by2) - jnp.maximum(y1, by1), 0.0)
                inter = jnp.where(invalid, -1.0, iw * ih)
                union = at + ap - inter
                upd = inter * b_u > b_i * union
                b_i = jnp.where(upd, inter, b_i)
                b_u = jnp.where(upd, union, b_u)
                b_t = jnp.where(upd, t, b_t)
                o2 = t * 3 * L
                li = mv[pl.ds(o2, L)]
                lu = mv[pl.ds(o2 + L, L)]
                lidx = lax.bitcast_convert_type(mv[pl.ds(o2 + 2 * L, L)],
                                                jnp.int32)
                upd2 = inter * lu > li * union
                mv[pl.ds(o2, L)] = jnp.where(upd2, inter, li)
                mv[pl.ds(o2 + L, L)] = jnp.where(upd2, union, lu)
                mv[pl.ds(o2 + 2 * L, L)] = lax.bitcast_convert_type(
                    jnp.where(upd2, gidx, lidx), jnp.float32)
            bi[sl] = b_i
            bu[sl] = b_u
            bti[sl] = b_t
            return 0

        lax.fori_loop(0, NCHUNK, chunk_body, 0)

    with jax.named_scope("sc_bp_force"):
        def bp_body(t, _):
            o2 = pl.multiple_of(t * 3 * L, L)
            li = mv[pl.ds(o2, L)]
            lu = mv[pl.ds(o2 + L, L)]
            lidx = lax.bitcast_convert_type(mv[pl.ds(o2 + 2 * L, L)],
                                            jnp.int32)
            q = li / lu
            m = jnp.max(q)
            cand = jnp.where(q == m, lidx, NP_PAD)
            bp[t] = jnp.min(cand)
            return 0

        lax.fori_loop(0, NOBJ, bp_body, 0)

        def force_body(t, _):
            j = bp[t]
            o = pl.multiple_of((j >> 4) << 4, L)
            lane = j & 15
            sl = pl.ds(o, L)
            hit = iot == lane
            bi[sl] = jnp.where(hit, 2.0, bi[sl])
            bu[sl] = jnp.where(hit, 1.0, bu[sl])
            bti[sl] = jnp.where(hit, t, bti[sl])
            return 0

        lax.fori_loop(0, NOBJ, force_body, 0)

        for j in range(L):
            hist[pl.ds(j * L, L)] = jnp.zeros((L,), jnp.int32)

    ones = jnp.full((L,), 1, jnp.int32)

    with jax.named_scope("sc_wait_lc"):
        for c in copies[4:10]:
            c.wait()

    with jax.named_scope("sc_passC"):
        def main_body(ci, carry):
            acc_l, acc_c, npos = carry
            o = pl.multiple_of(ci * L, L)
            sl = pl.ds(o, L)
            pos = 2.0 * bi[sl] >= bu[sl]
            bt = bti[sl]
            mx1 = plsc.load_gather(tx1, [bt])
            my1 = plsc.load_gather(ty1, [bt])
            mx2 = plsc.load_gather(tx2, [bt])
            my2 = plsc.load_gather(ty2, [bt])
            x1, y1, x2, y2 = px1[sl], py1[sl], px2[sl], py2[sl]
            cx = (x1 + x2) * 0.5
            cy = (y1 + y2) * 0.5
            w = x2 - x1
            h = y2 - y1
            gcx = ((mx1 + mx2) * 0.5 - cx) / (0.1 * w)
            gcy = ((my1 + my2) * 0.5 - cy) / (0.1 * h)
            gw = _ln16((mx2 - mx1) / w) * 5.0
            gh = _ln16((my2 - my1) / h) * 5.0
            sm = jnp.zeros((L,), jnp.float32)
            for lref, g in ((l0, gcx), (l1, gcy), (l2, gw), (l3, gh)):
                d = lref[sl] - g
                ad = jnp.abs(d)
                sm = sm + jnp.where(ad < 1.0, 0.5 * d * d, ad - 0.5)
            acc_l = acc_l + jnp.where(pos, sm, 0.0)
            npos = npos + jnp.where(pos, 1, 0)
            x0 = c0[sl]
            xo = c1[sl]
            mxx = jnp.maximum(x0, xo)
            mnn = jnp.minimum(x0, xo)
            lse = _ln16(1.0 + jnp.exp(mnn - mxx)) + mxx
            xt = jnp.where(pos, xo, x0)
            ce = lse - xt
            acc_c = acc_c + jnp.where(pos, ce, 0.0)
            gidx = o + iot
            keyf = jnp.where(jnp.logical_and(gidx < NP, jnp.logical_not(pos)),
                             ce, 0.0)
            kb = lax.bitcast_convert_type(keyf, jnp.int32)
            keyr[sl] = kb
            plsc.addupdate_scatter(hist, [(kb >> 24) & 255], ones)
            return acc_l, acc_c, npos

        acc_l, acc_c, npos = lax.fori_loop(
            0, NCHUNK, main_body,
            (jnp.zeros((L,), jnp.float32), jnp.zeros((L,), jnp.float32),
             jnp.zeros((L,), jnp.int32)))
        num_pos = jnp.sum(npos)
        loss_l_s = jnp.sum(acc_l)
        ce_pos_s = jnp.sum(acc_c)
        k = jnp.minimum(NEGPOS_RATIO * num_pos, NP - 1)

    with jax.named_scope("sc_radix"):
        UNROLL = 6
        prefix = jnp.int32(0)
        c_before = jnp.int32(0)
        for rnd, d in enumerate((24, 16, 8, 0)):
            if rnd > 0:
                for j in range(L):
                    hist[pl.ds(j * L, L)] = jnp.zeros((L,), jnp.int32)

                def hsweep(cg, _, d=d, prefix=prefix):
                    o0 = pl.multiple_of(cg * (L * UNROLL), L)
                    for u in range(UNROLL):
                        kb = keyr[pl.ds(o0 + u * L, L)]
                        digit = (kb >> d) & 255
                        region = (kb >> (d + 8)) == prefix
                        plsc.addupdate_scatter(hist, [digit], ones,
                                               mask=region)
                    return 0

                lax.fori_loop(0, NCHUNK // UNROLL, hsweep, 0)

            csum = jnp.zeros((L,), jnp.int32)
            for c in range(L):
                s = jnp.sum(hist[pl.ds(c * L, L)])
                csum = jnp.where(iot == c, s, csum)
            rs = jnp.flip(csum, axis=0)
            cs = plsc.cumsum(rs)
            k_rel = k - c_before
            jj = _scalar(plsc.all_reduce_ffs(cs >= k_rel))
            gt_chunks = jnp.sum(jnp.where(iot < jj, rs, 0))
            cstar = 15 - jj
            hc = hist[pl.ds(pl.multiple_of(cstar * L, L), L)]
            rhc = jnp.flip(hc, axis=0)
            cs2 = plsc.cumsum(rhc)
            jj2 = _scalar(plsc.all_reduce_ffs(cs2 >= (k_rel - gt_chunks)))
            gt_in = jnp.sum(jnp.where(iot < jj2, rhc, 0))
            digit = cstar * L + (15 - jj2)
            c_before = c_before + gt_chunks + gt_in
            prefix = digit if rnd == 0 else (prefix << 8) | digit

        v = prefix
        c_gt = c_before

    with jax.named_scope("sc_passE"):
        def esweep(cg, acc):
            o0 = pl.multiple_of(cg * (L * UNROLL), L)
            for u in range(UNROLL):
                kb = keyr[pl.ds(o0 + u * L, L)]
                keyf = lax.bitcast_convert_type(kb, jnp.float32)
                acc = acc + jnp.where(kb > v, keyf, 0.0)
            return acc

        acc_n = lax.fori_loop(0, NCHUNK // UNROLL, esweep,
                              jnp.zeros((L,), jnp.float32))
        vf = lax.bitcast_convert_type(v, jnp.float32)
        loss_c_s = (ce_pos_s + jnp.sum(acc_n)
                    + (k - c_gt).astype(jnp.float32) * vf)

        outv[...] = jnp.where(
            iot == 0, loss_l_s,
            jnp.where(iot == 1, loss_c_s,
                      jnp.where(iot == 2, num_pos.astype(jnp.float32), 0.0)))
        pltpu.sync_copy(outv, out_hbm.at[wid])


@functools.partial(
    pl.kernel,
    mesh=plsc.VectorSubcoreMesh(core_axis_name="c", subcore_axis_name="s"),
    out_type=jax.ShapeDtypeStruct((NUM, L), jnp.float32),
    compiler_params=pltpu.CompilerParams(needs_layout_passes=False),
    scratch_types=[
        pltpu.VMEM((NP_PAD,), jnp.float32),
        pltpu.VMEM((NP_PAD,), jnp.float32),
        pltpu.VMEM((NP_PAD,), jnp.float32),
        pltpu.VMEM((NP_PAD,), jnp.float32),
        pltpu.VMEM((NP_PAD,), jnp.float32),
        pltpu.VMEM((NP_PAD,), jnp.float32),
        pltpu.VMEM((NP_PAD,), jnp.float32),
        pltpu.VMEM((NP_PAD,), jnp.float32),
        pltpu.VMEM((NP_PAD,), jnp.float32),
        pltpu.VMEM((NP_PAD,), jnp.float32),
        pltpu.VMEM((NP_PAD,), jnp.float32),
        pltpu.VMEM((NP_PAD,), jnp.float32),
        pltpu.VMEM((NP_PAD,), jnp.int32),
        pltpu.VMEM((NP_PAD,), jnp.int32),
        pltpu.VMEM((32,), jnp.float32),
        pltpu.VMEM((32,), jnp.float32),
        pltpu.VMEM((32,), jnp.float32),
        pltpu.VMEM((32,), jnp.float32),
        pltpu.VMEM((NOBJ * 5 * L,), jnp.float32),
        pltpu.VMEM((NOBJ * 3 * L,), jnp.float32),
        pltpu.VMEM((256,), jnp.int32),
        pltpu.VMEM((L,), jnp.float32),
        pltpu.SMEM((32,), jnp.int32),
        pltpu.SemaphoreType.DMA,
    ],
)
def _sc_loss(loc_hbm, conf_hbm, pri_hbm, tr_hbm, out_hbm, *scratch):
    _sc_body(loc_hbm, conf_hbm, pri_hbm, tr_hbm, out_hbm, *scratch)


def kernel(loc_data, conf_data, priors, targets):
    num, np_, _ = loc_data.shape
    pad = NP_PAD - np_
    loc_r = jnp.transpose(loc_data, (0, 2, 1))
    loc_r = jnp.pad(loc_r, ((0, 0), (0, 0), (0, pad))).reshape(num * 4, NP_PAD)
    conf_r = jnp.transpose(conf_data, (0, 2, 1))
    conf_r = jnp.pad(conf_r, ((0, 0), (0, 0), (0, pad))).reshape(num * 2, NP_PAD)
    pri_r = jnp.transpose(priors[:np_], (1, 0))
    pri_r = jnp.pad(pri_r, ((0, 0), (0, pad)), constant_values=1.0)
    tr_r = jnp.transpose(targets[:, :, :4], (0, 2, 1))
    tr_r = jnp.pad(tr_r, ((0, 0), (0, 0), (0, 32 - tr_r.shape[2])))
    tr_r = tr_r.reshape(num * 4, 32)
    part = _sc_loss(loc_r, conf_r, pri_r, tr_r)
    n = jnp.sum(part[:, 2])
    return jnp.sum(part[:, 0]) / n, jnp.sum(part[:, 1]) / n
